# Initial kernel scaffold; baseline (speedup 1.0000x reference)
import jax
import jax.numpy as jnp
from jax.experimental import pallas as pl


def kernel(u, mem_keys, mem_values, W_proj_r, cb_r, W_qkey, W_out):
    raise NotImplementedError("write your pallas kernel here")


# speedup vs baseline: 3.6359x; 3.6359x over previous
import functools

import jax
import jax.numpy as jnp
from jax import lax
from jax.experimental import pallas as pl
from jax.experimental.pallas import tpu as pltpu
from jax.experimental.pallas import tpu_sc as plsc

_B, _T, _D = 4, 2048, 2048
_H, _G, _Kc, _gd = 4, 2, 256, 16
_BUCKETS, _ASSOC, _KEY_DIM, _MEM_DIM = 65536, 1, 128, 256
_N = _B * _T
_HG = _H * _G
_BT = 512
_NBLK = _N // _BT
_NH = _N * _H

_NW = 32
_M = _NH // _NW
_C = 128
_NCH = _M // _C


def _route_body(u_ref, wc_ref, cb_ref, e2_ref, idx_ref, qk_ref):
    u = u_ref[...]
    z = jnp.dot(u, wc_ref[...], preferred_element_type=jnp.float32)
    y = z[:, : _HG * _gd]
    qk_ref[...] = z[:, _HG * _gd :]
    cols = []
    for h in range(_H):
        acc = None
        for g in range(_G):
            hg = h * _G + g
            yhg = y[:, hg * _gd : (hg + 1) * _gd]
            dot = lax.dot_general(
                yhg, cb_ref[hg],
                dimension_numbers=(((1,), (1,)), ((), ())),
                preferred_element_type=jnp.float32,
            )
            y2 = jnp.sum(yhg * yhg, axis=1, keepdims=True)
            dist = y2 + e2_ref[hg : hg + 1, :] - 2.0 * dot
            m = jnp.min(dist, axis=1, keepdims=True)
            kiota = lax.broadcasted_iota(jnp.int32, (_BT, _Kc), 1)
            best = jnp.min(jnp.where(dist <= m, kiota, _Kc), axis=1, keepdims=True)
            contrib = best * (_Kc ** g)
            acc = contrib if acc is None else acc + contrib
        cols.append(acc)
    idx_ref[...] = jnp.concatenate(cols, axis=1)


_route = pl.pallas_call(
    _route_body,
    grid=(_NBLK,),
    in_specs=[
        pl.BlockSpec((_BT, _D), lambda i: (i, 0)),
        pl.BlockSpec((_D, _HG * _gd + _KEY_DIM), lambda i: (0, 0)),
        pl.BlockSpec((_HG, _Kc, _gd), lambda i: (0, 0, 0)),
        pl.BlockSpec((_HG, _Kc), lambda i: (0, 0)),
    ],
    out_specs=[
        pl.BlockSpec((_BT, _H), lambda i: (i, 0)),
        pl.BlockSpec((_BT, _KEY_DIM), lambda i: (i, 0)),
    ],
    out_shape=[
        jax.ShapeDtypeStruct((_N, _H), jnp.int32),
        jax.ShapeDtypeStruct((_N, _KEY_DIM), jnp.float32),
    ],
)


def _gather_body(keys_hbm, vals_hbm, idx_hbm, keys_out, vals_out,
                 idx_v, kbuf, vbuf, ksem, vsem):
    wid = lax.axis_index("s") * 2 + lax.axis_index("c")
    base = wid * _M
    pltpu.sync_copy(idx_hbm.at[wid], idx_v)
    for j in range(_NCH):
        kcp = pltpu.async_copy(keys_hbm.at[idx_v.at[j]], kbuf, ksem)
        vcp = pltpu.async_copy(vals_hbm.at[idx_v.at[j]], vbuf, vsem)
        kcp.wait()
        vcp.wait()
        pltpu.sync_copy(kbuf, keys_out.at[pl.ds(base + j * _C, _C)])
        pltpu.sync_copy(vbuf, vals_out.at[pl.ds(base + j * _C, _C)])


@functools.lru_cache(maxsize=1)
def _make_gather():
    return functools.partial(
        pl.kernel,
        mesh=plsc.VectorSubcoreMesh(core_axis_name="c", subcore_axis_name="s"),
        out_type=[
            jax.ShapeDtypeStruct((_NH, _KEY_DIM), jnp.float32),
            jax.ShapeDtypeStruct((_NH, _MEM_DIM), jnp.float32),
        ],
        scratch_types=[
            pltpu.VMEM((_NCH, _C), jnp.int32),
            pltpu.VMEM((_C, _KEY_DIM), jnp.float32),
            pltpu.VMEM((_C, _MEM_DIM), jnp.float32),
            pltpu.SemaphoreType.DMA,
            pltpu.SemaphoreType.DMA,
        ],
    )(_gather_body)


def _read_body(qk_ref, keys_ref, vals_ref, wt_ref, out_ref):
    qk = qk_ref[...]
    scores = []
    for h in range(_H):
        s = jnp.sum(qk * keys_ref[:, h, :], axis=1, keepdims=True)
        scores.append(s)
    s = jnp.concatenate(scores, axis=1) / jnp.sqrt(jnp.float32(_KEY_DIM))
    m = jnp.max(s, axis=1, keepdims=True)
    e = jnp.exp(s - m)
    a = e / jnp.sum(e, axis=1, keepdims=True)
    read = None
    for h in range(_H):
        contrib = a[:, h : h + 1] * vals_ref[:, h, :]
        read = contrib if read is None else read + contrib
    out_ref[...] = jnp.dot(read, wt_ref[...], preferred_element_type=jnp.float32)


_read = pl.pallas_call(
    _read_body,
    grid=(_NBLK,),
    in_specs=[
        pl.BlockSpec((_BT, _KEY_DIM), lambda i: (i, 0)),
        pl.BlockSpec((_BT, _H, _KEY_DIM), lambda i: (i, 0, 0)),
        pl.BlockSpec((_BT, _H, _MEM_DIM), lambda i: (i, 0, 0)),
        pl.BlockSpec((_MEM_DIM, _D), lambda i: (0, 0)),
    ],
    out_specs=pl.BlockSpec((_BT, _D), lambda i: (i, 0)),
    out_shape=jax.ShapeDtypeStruct((_N, _D), jnp.float32),
)


def kernel(u, mem_keys, mem_values, W_proj_r, cb_r, W_qkey, W_out):
    u2 = u.reshape(_N, _D)
    wc = jnp.concatenate([W_proj_r.T, W_qkey.T], axis=1)
    cb2 = cb_r.reshape(_HG, _Kc, _gd)
    e2 = jnp.sum(cb_r * cb_r, axis=-1).reshape(_HG, _Kc)
    idx, qk = _route(u2, wc, cb2, e2)
    idx3 = idx.reshape(_NW, _NCH, _C)
    keys_sel, vals_sel = _make_gather()(
        mem_keys.reshape(_BUCKETS, _KEY_DIM),
        mem_values.reshape(_BUCKETS, _MEM_DIM),
        idx3,
    )
    out2 = _read(
        qk,
        keys_sel.reshape(_N, _H, _KEY_DIM),
        vals_sel.reshape(_N, _H, _MEM_DIM),
        W_out.T,
    )
    return out2.reshape(_B, _T, _D)
